# baseline (speedup 1.0000x reference)
import math

import jax
import jax.numpy as jnp
from jax import lax
from jax.experimental import pallas as pl
from jax.experimental.pallas import tpu as pltpu
from jax.experimental.pallas import tpu_sc as plsc

_V = 100000
_N = 1024
_PAD = 0
_SMOOTH = 0.1
_CONF = 1.0 - _SMOOTH
_FILL = _SMOOTH / (_V - 2)
_C = _FILL * math.log(_FILL) * (_V - 2) + _CONF * math.log(_CONF)

_NSC = 512
_NW = 32
_RPW = 16
_NG = _NSC // (_NW * _RPW)
_RB = 32

_CH = 26 * 128
_NCH = 30
_TAIL0 = _NCH * _CH
_TAILW = 128
_SC_COLS = _TAIL0 + _TAILW
_REM = _V - _SC_COLS


def _sc_kernel(x_hbm, tgt_hbm, rows_out, gath_out,
               buf0, buf1, tbuf, tgt_v, res_v, gbuf, gres_v,
               sem0, sem1, tsem, gsem):
    wid = lax.axis_index("s") * 2 + lax.axis_index("c")
    lanes = lax.iota(jnp.int32, 16)
    m0 = (1 - jnp.minimum(lanes, 1)).astype(jnp.float32)

    acc_g = jnp.zeros((16,), jnp.float32)
    for grp in range(_NG):
        base = grp * (_NW * _RPW) + wid * _RPW

        pltpu.sync_copy(tgt_hbm.at[pl.ds(base, _RPW)], tgt_v)
        tv = tgt_v[...]

        tvals = []
        c0s = []
        for r in range(_RPW):
            t = tv[r]
            c0 = jnp.minimum((t // 128) * 128, _TAIL0)
            tvals.append(t)
            c0s.append(c0)
            pltpu.async_copy(
                x_hbm.at[pl.ds(base + (r // 8) * 8, 8), pl.ds(c0, 128)],
                gbuf.at[r], gsem)

        tail_cp = pltpu.async_copy(
            x_hbm.at[pl.ds(base, _RPW), pl.ds(_TAIL0, _TAILW)], tbuf, tsem)
        pltpu.async_copy(x_hbm.at[pl.ds(base, _RPW), pl.ds(0, _CH)],
                         buf0, sem0)
        accs = tuple(jnp.zeros((16,), jnp.float32) for _ in range(_RPW))
        for c in range(_NCH):
            cur, cur_sem = (buf0, sem0) if c % 2 == 0 else (buf1, sem1)
            nxt, nxt_sem = (buf1, sem1) if c % 2 == 0 else (buf0, sem0)
            if c + 1 < _NCH:
                pltpu.async_copy(
                    x_hbm.at[pl.ds(base, _RPW), pl.ds((c + 1) * _CH, _CH)],
                    nxt, nxt_sem)
            pltpu.make_async_copy(
                x_hbm.at[pl.ds(base, _RPW), pl.ds(c * _CH, _CH)],
                cur, cur_sem).wait()

            def body(i, a, _cur=cur):
                return tuple(a[r] + _cur[r, pl.ds(i * 16, 16)]
                             for r in range(_RPW))
            accs = lax.fori_loop(0, _CH // 16, body, accs)
            if c == 0:
                accs = tuple(
                    accs[r] - cur[r, pl.ds(0, 16)] * m0
                    for r in range(_RPW))

        tail_cp.wait()
        for i in range(_TAILW // 16):
            accs = tuple(accs[r] + tbuf[r, pl.ds(i * 16, 16)]
                         for r in range(_RPW))
        for r in range(_RPW):
            res_v[r] = accs[r]
        pltpu.sync_copy(res_v, rows_out.at[pl.ds(base, _RPW)])

        for r in range(_RPW):
            pltpu.make_async_copy(
                x_hbm.at[pl.ds(base + (r // 8) * 8, 8), pl.ds(c0s[r], 128)],
                gbuf.at[r], gsem).wait()
        for r in range(_RPW):
            t = tvals[r]
            d = ((t % 128) // 16) * 16
            v = gbuf[r, r % 8, pl.ds(d, 16)]
            eq = (1 - jnp.minimum(jnp.abs(lanes - t % 16), 1)
                  ).astype(jnp.float32)
            w = jnp.where((t != _PAD) & (t < _SC_COLS), 1.0, 0.0)
            acc_g = acc_g + v * (eq * w)

    gres_v[...] = acc_g
    pltpu.sync_copy(gres_v, gath_out.at[wid])


def _tc_stream_kernel(x_ref, tgt_ref, out_ref):
    i = pl.program_id(0)
    x = x_ref[...]
    tgt = tgt_ref[...]
    rowsum = jnp.sum(x, axis=1, keepdims=True)
    cols = lax.broadcasted_iota(jnp.int32, x.shape, 1)
    xt = jnp.sum(jnp.where(cols == tgt, x, 0.0), axis=1, keepdims=True)
    x0 = x[:, 0:1]
    part = jnp.sum(jnp.where(
        tgt != _PAD,
        _C - _FILL * (rowsum - x0) + (_FILL - _CONF) * xt, 0.0))

    @pl.when(i == 0)
    def _():
        out_ref[0, 0] = 0.0

    out_ref[0, 0] += part


def _combine_kernel(rows_ref, gath_ref, tgt_ref, xtail_ref, tcp_ref, out_ref):
    xtail = xtail_ref[...]
    srow = (jnp.sum(rows_ref[...], axis=1, keepdims=True)
            + jnp.sum(xtail, axis=1, keepdims=True))
    tgt = tgt_ref[...]
    valid = tgt != _PAD
    s_row = jnp.sum(jnp.where(valid, _C - _FILL * srow, 0.0))
    cols = lax.broadcasted_iota(jnp.int32, xtail.shape, 1) + _SC_COLS
    xt_tail = jnp.sum(jnp.where(tgt == cols, xtail, 0.0))
    s_g = (_FILL - _CONF) * (jnp.sum(gath_ref[...]) + xt_tail)
    out_ref[0, 0] = s_row + s_g + tcp_ref[0, 0]


def kernel(x, target):
    tgt2 = target.reshape(_N, 1)

    x_sc = lax.slice(x, (0, 0), (_NSC, _V))

    sc_rows, sc_gath = pl.kernel(
        _sc_kernel,
        out_type=(
            jax.ShapeDtypeStruct((_NSC, 16), jnp.float32),
            jax.ShapeDtypeStruct((_NW, 16), jnp.float32),
        ),
        mesh=plsc.VectorSubcoreMesh(core_axis_name="c", subcore_axis_name="s"),
        compiler_params=pltpu.CompilerParams(use_tc_tiling_on_sc=True),
        scratch_types=[
            pltpu.VMEM((_RPW, _CH), jnp.float32),
            pltpu.VMEM((_RPW, _CH), jnp.float32),
            pltpu.VMEM((_RPW, _TAILW), jnp.float32),
            pltpu.VMEM((16,), jnp.int32),
            pltpu.VMEM((_RPW, 16), jnp.float32),
            pltpu.VMEM((_RPW, 8, 128), jnp.float32),
            pltpu.VMEM((16,), jnp.float32),
            pltpu.SemaphoreType.DMA,
            pltpu.SemaphoreType.DMA,
            pltpu.SemaphoreType.DMA,
            pltpu.SemaphoreType.DMA,
        ],
    )(x_sc, target)

    tc_part = pl.pallas_call(
        _tc_stream_kernel,
        grid=((_N - _NSC) // _RB,),
        in_specs=[
            pl.BlockSpec((_RB, _V), lambda i: (_NSC // _RB + i, 0)),
            pl.BlockSpec((_RB, 1), lambda i: (_NSC // _RB + i, 0)),
        ],
        out_specs=pl.BlockSpec((1, 1), lambda i: (0, 0),
                               memory_space=pltpu.SMEM),
        out_shape=jax.ShapeDtypeStruct((1, 1), jnp.float32),
    )(x, tgt2)

    xtail = lax.slice(x, (0, _SC_COLS), (_NSC, _V))
    tgt_sc = lax.slice(tgt2, (0, 0), (_NSC, 1))

    out = pl.pallas_call(
        _combine_kernel,
        in_specs=[
            pl.BlockSpec(memory_space=pltpu.VMEM),
            pl.BlockSpec(memory_space=pltpu.VMEM),
            pl.BlockSpec(memory_space=pltpu.VMEM),
            pl.BlockSpec(memory_space=pltpu.VMEM),
            pl.BlockSpec(memory_space=pltpu.SMEM),
        ],
        out_specs=pl.BlockSpec(memory_space=pltpu.SMEM),
        out_shape=jax.ShapeDtypeStruct((1, 1), jnp.float32),
    )(sc_rows, sc_gath, tgt_sc, xtail, tc_part)

    return out[0, 0]


# speedup vs baseline: 1.2697x; 1.2697x over previous
import math

import jax
import jax.numpy as jnp
from jax import lax
from jax.experimental import pallas as pl
from jax.experimental.pallas import tpu as pltpu
from jax.experimental.pallas import tpu_sc as plsc

_V = 100000
_N = 1024
_PAD = 0
_SMOOTH = 0.1
_CONF = 1.0 - _SMOOTH
_FILL = _SMOOTH / (_V - 2)
_C = _FILL * math.log(_FILL) * (_V - 2) + _CONF * math.log(_CONF)

_NSC = 512
_NW = 32
_RPW = 16
_NG = _NSC // (_NW * _RPW)
_RB = 32

_CH = 26 * 128
_NCH = 30
_TAIL0 = _NCH * _CH
_TAILW = 128
_SC_COLS = _TAIL0 + _TAILW
_REM = _V - _SC_COLS


def _sc_kernel(x_hbm, tgt_hbm, rows_out, gath_out,
               buf0, buf1, tbuf, tgt_v, res_v, gbuf, gres_v,
               sem0, sem1, tsem, gsem):
    wid = lax.axis_index("s") * 2 + lax.axis_index("c")
    lanes = lax.iota(jnp.int32, 16)
    m0 = (1 - jnp.minimum(lanes, 1)).astype(jnp.float32)

    acc_g = jnp.zeros((16,), jnp.float32)
    for grp in range(_NG):
        base = grp * (_NW * _RPW) + wid * _RPW

        pltpu.sync_copy(tgt_hbm.at[pl.ds(base, _RPW)], tgt_v)
        tv = tgt_v[...]

        tvals = []
        c0s = []
        for r in range(_RPW):
            t = tv[r]
            c0 = jnp.minimum((t // 128) * 128, _TAIL0)
            tvals.append(t)
            c0s.append(c0)
            pltpu.async_copy(
                x_hbm.at[pl.ds(base + (r // 8) * 8, 8), pl.ds(c0, 128)],
                gbuf.at[r], gsem)

        tail_cp = pltpu.async_copy(
            x_hbm.at[pl.ds(base, _RPW), pl.ds(_TAIL0, _TAILW)], tbuf, tsem)
        pltpu.async_copy(x_hbm.at[pl.ds(base, _RPW), pl.ds(0, _CH)],
                         buf0, sem0)
        accs = tuple(jnp.zeros((16,), jnp.float32) for _ in range(_RPW))
        for c in range(_NCH):
            cur, cur_sem = (buf0, sem0) if c % 2 == 0 else (buf1, sem1)
            nxt, nxt_sem = (buf1, sem1) if c % 2 == 0 else (buf0, sem0)
            if c + 1 < _NCH:
                pltpu.async_copy(
                    x_hbm.at[pl.ds(base, _RPW), pl.ds((c + 1) * _CH, _CH)],
                    nxt, nxt_sem)
            pltpu.make_async_copy(
                x_hbm.at[pl.ds(base, _RPW), pl.ds(c * _CH, _CH)],
                cur, cur_sem).wait()

            def body(i, a, _cur=cur):
                return tuple(a[r] + _cur[r, pl.ds(i * 16, 16)]
                             for r in range(_RPW))
            accs = lax.fori_loop(0, _CH // 16, body, accs)
            if c == 0:
                accs = tuple(
                    accs[r] - cur[r, pl.ds(0, 16)] * m0
                    for r in range(_RPW))

        tail_cp.wait()
        for i in range(_TAILW // 16):
            accs = tuple(accs[r] + tbuf[r, pl.ds(i * 16, 16)]
                         for r in range(_RPW))
        for r in range(_RPW):
            res_v[r] = accs[r]
        pltpu.sync_copy(res_v, rows_out.at[pl.ds(base, _RPW)])

        for r in range(_RPW):
            pltpu.make_async_copy(
                x_hbm.at[pl.ds(base + (r // 8) * 8, 8), pl.ds(c0s[r], 128)],
                gbuf.at[r], gsem).wait()
        for r in range(_RPW):
            t = tvals[r]
            d = ((t % 128) // 16) * 16
            v = gbuf[r, r % 8, pl.ds(d, 16)]
            eq = (1 - jnp.minimum(jnp.abs(lanes - t % 16), 1)
                  ).astype(jnp.float32)
            w = jnp.where((t != _PAD) & (t < _SC_COLS), 1.0, 0.0)
            acc_g = acc_g + v * (eq * w)

    gres_v[...] = acc_g
    pltpu.sync_copy(gres_v, gath_out.at[wid])


def _tc_stream_kernel(x_ref, tgt_ref, out_ref):
    i = pl.program_id(0)
    x = x_ref[...]
    tgt = tgt_ref[...]
    rowsum = jnp.sum(x, axis=1, keepdims=True)
    cols = lax.broadcasted_iota(jnp.int32, x.shape, 1)
    xt = jnp.sum(jnp.where(cols == tgt, x, 0.0), axis=1, keepdims=True)
    x0 = x[:, 0:1]
    part = jnp.sum(jnp.where(
        tgt != _PAD,
        _C - _FILL * (rowsum - x0) + (_FILL - _CONF) * xt, 0.0))

    @pl.when(i == 0)
    def _():
        out_ref[0, 0] = 0.0

    out_ref[0, 0] += part


def _combine_kernel(rows_ref, gath_ref, tgt_ref, xtail_ref, tcp_ref, out_ref):
    xtail = xtail_ref[...]
    srow = (jnp.sum(rows_ref[...], axis=1, keepdims=True)
            + jnp.sum(xtail, axis=1, keepdims=True))
    tgt = tgt_ref[...]
    valid = tgt != _PAD
    s_row = jnp.sum(jnp.where(valid, _C - _FILL * srow, 0.0))
    cols = lax.broadcasted_iota(jnp.int32, xtail.shape, 1) + _SC_COLS
    xt_tail = jnp.sum(jnp.where(tgt == cols, xtail, 0.0))
    s_g = (_FILL - _CONF) * (jnp.sum(gath_ref[...]) + xt_tail)
    out_ref[0, 0] = s_row + s_g + tcp_ref[0, 0]


def kernel(x, target):
    tgt2 = target.reshape(_N, 1)

    sc_rows, sc_gath = pl.kernel(
        _sc_kernel,
        out_type=(
            jax.ShapeDtypeStruct((_NSC, 16), jnp.float32),
            jax.ShapeDtypeStruct((_NW, 16), jnp.float32),
        ),
        mesh=plsc.VectorSubcoreMesh(core_axis_name="c", subcore_axis_name="s"),
        compiler_params=pltpu.CompilerParams(use_tc_tiling_on_sc=True),
        scratch_types=[
            pltpu.VMEM((_RPW, _CH), jnp.float32),
            pltpu.VMEM((_RPW, _CH), jnp.float32),
            pltpu.VMEM((_RPW, _TAILW), jnp.float32),
            pltpu.VMEM((16,), jnp.int32),
            pltpu.VMEM((_RPW, 16), jnp.float32),
            pltpu.VMEM((_RPW, 8, 128), jnp.float32),
            pltpu.VMEM((16,), jnp.float32),
            pltpu.SemaphoreType.DMA,
            pltpu.SemaphoreType.DMA,
            pltpu.SemaphoreType.DMA,
            pltpu.SemaphoreType.DMA,
        ],
    )(x, target)

    tc_part = pl.pallas_call(
        _tc_stream_kernel,
        grid=((_N - _NSC) // _RB,),
        in_specs=[
            pl.BlockSpec((_RB, _V), lambda i: (_NSC // _RB + i, 0)),
            pl.BlockSpec((_RB, 1), lambda i: (_NSC // _RB + i, 0)),
        ],
        out_specs=pl.BlockSpec((1, 1), lambda i: (0, 0),
                               memory_space=pltpu.SMEM),
        out_shape=jax.ShapeDtypeStruct((1, 1), jnp.float32),
    )(x, tgt2)

    xtail = lax.slice(x, (0, _SC_COLS), (_NSC, _V))
    tgt_sc = lax.slice(tgt2, (0, 0), (_NSC, 1))

    out = pl.pallas_call(
        _combine_kernel,
        in_specs=[
            pl.BlockSpec(memory_space=pltpu.VMEM),
            pl.BlockSpec(memory_space=pltpu.VMEM),
            pl.BlockSpec(memory_space=pltpu.VMEM),
            pl.BlockSpec(memory_space=pltpu.VMEM),
            pl.BlockSpec(memory_space=pltpu.SMEM),
        ],
        out_specs=pl.BlockSpec(memory_space=pltpu.SMEM),
        out_shape=jax.ShapeDtypeStruct((1, 1), jnp.float32),
    )(sc_rows, sc_gath, tgt_sc, xtail, tc_part)

    return out[0, 0]
